# Initial kernel scaffold; baseline (speedup 1.0000x reference)
import jax
import jax.numpy as jnp
from jax.experimental import pallas as pl


def kernel(x, active_mask, W_r, expert_bias):
    raise NotImplementedError("write your pallas kernel here")


# speedup vs baseline: 4.6644x; 4.6644x over previous
import jax
import jax.numpy as jnp
from jax.experimental import pallas as pl
from jax.experimental.pallas import tpu as pltpu

_K = 8


def _router_kernel(x_ref, w_ref, bias_ref, act_ref,
                   sel_ref, probs_ref, loss_ref, vio_ref,
                   counts_scr, act_scr):
    i = pl.program_id(0)
    nsteps = pl.num_programs(0)

    @pl.when(i == 0)
    def _init():
        counts_scr[...] = jnp.zeros_like(counts_scr)
        act_scr[...] = jnp.zeros_like(act_scr)

    x = x_ref[...]
    w = w_ref[...]
    logits = jnp.dot(x, w, preferred_element_type=jnp.float32)
    bias = bias_ref[...]
    T, L = logits.shape
    iota = jax.lax.broadcasted_iota(jnp.int32, (T, L), 1)
    neg_inf = jnp.float32(-jnp.inf)

    b = logits + bias
    sel_cols = []
    val_cols = []
    onehot_sum = jnp.zeros((T, L), jnp.float32)
    for _ in range(_K):
        m = jnp.max(b, axis=1, keepdims=True)
        idx = jnp.min(jnp.where(b == m, iota, L), axis=1, keepdims=True)
        onehot = iota == idx
        val = jnp.max(jnp.where(onehot, logits, neg_inf), axis=1,
                      keepdims=True)
        sel_cols.append(idx)
        val_cols.append(val)
        onehot_sum = onehot_sum + onehot.astype(jnp.float32)
        b = jnp.where(onehot, neg_inf, b)

    sel = jnp.concatenate(sel_cols, axis=1)
    vals = jnp.concatenate(val_cols, axis=1)
    mx = jnp.max(vals, axis=1, keepdims=True)
    e = jnp.exp(vals - mx)
    probs = e / jnp.sum(e, axis=1, keepdims=True)

    sel_ref[...] = sel
    probs_ref[...] = probs

    act = act_ref[...]
    counts_scr[...] += jnp.sum(onehot_sum * act, axis=0, keepdims=True)
    act_scr[...] += jnp.sum(act, axis=(0, 1), keepdims=True)

    @pl.when(i == nsteps - 1)
    def _finish():
        counts = counts_scr[...]
        total = act_scr[...] * jnp.float32(_K)
        freqs = counts / total
        loss_ref[...] = jnp.sum(bias * freqs, axis=1, keepdims=True)
        vio_ref[...] = jnp.float32(L) * jnp.max(freqs - 1.0 / L, axis=1,
                                                keepdims=True)


def kernel(x, active_mask, W_r, expert_bias):
    Bb, Nn, Hh = x.shape
    L = W_r.shape[1]
    BN = Bb * Nn
    T = 512
    xf = x.reshape(BN, Hh)
    act = active_mask.reshape(BN, 1).astype(jnp.float32)
    bias2 = expert_bias.reshape(1, L)

    out_shape = [
        jax.ShapeDtypeStruct((BN, _K), jnp.int32),
        jax.ShapeDtypeStruct((BN, _K), jnp.float32),
        jax.ShapeDtypeStruct((1, 1), jnp.float32),
        jax.ShapeDtypeStruct((1, 1), jnp.float32),
    ]
    sel, probs, loss, vio = pl.pallas_call(
        _router_kernel,
        grid=(BN // T,),
        in_specs=[
            pl.BlockSpec((T, Hh), lambda i: (i, 0)),
            pl.BlockSpec((Hh, L), lambda i: (0, 0)),
            pl.BlockSpec((1, L), lambda i: (0, 0)),
            pl.BlockSpec((T, 1), lambda i: (i, 0)),
        ],
        out_specs=[
            pl.BlockSpec((T, _K), lambda i: (i, 0)),
            pl.BlockSpec((T, _K), lambda i: (i, 0)),
            pl.BlockSpec((1, 1), lambda i: (0, 0)),
            pl.BlockSpec((1, 1), lambda i: (0, 0)),
        ],
        out_shape=out_shape,
        scratch_shapes=[
            pltpu.VMEM((1, L), jnp.float32),
            pltpu.VMEM((1, 1), jnp.float32),
        ],
    )(xf, W_r, bias2, act)

    return (sel.reshape(Bb, Nn, _K), probs.reshape(Bb, Nn, _K),
            loss[0, 0], vio[0, 0])


# speedup vs baseline: 4.8609x; 1.0421x over previous
import jax
import jax.numpy as jnp
from jax.experimental import pallas as pl
from jax.experimental.pallas import tpu as pltpu

_K = 8


def _router_kernel(x_ref, w_ref, bias_ref, act_ref,
                   sel_ref, probs_ref, loss_ref, vio_ref,
                   counts_scr, act_scr):
    i = pl.program_id(0)
    nsteps = pl.num_programs(0)

    @pl.when(i == 0)
    def _init():
        counts_scr[...] = jnp.zeros_like(counts_scr)
        act_scr[...] = jnp.zeros_like(act_scr)

    x = x_ref[...]
    w = w_ref[...]
    logits = jnp.dot(x, w, preferred_element_type=jnp.float32)
    bias = bias_ref[...]
    T, L = logits.shape
    iota = jax.lax.broadcasted_iota(jnp.int32, (T, L), 1)
    neg_inf = jnp.float32(-jnp.inf)

    b = logits + bias
    sel_cols = []
    val_cols = []
    onehot_sum = jnp.zeros((T, L), jnp.float32)
    for _ in range(_K):
        m = jnp.max(b, axis=1, keepdims=True)
        idx = jnp.min(jnp.where(b == m, iota, L), axis=1, keepdims=True)
        onehot = iota == idx
        val = jnp.max(jnp.where(onehot, logits, neg_inf), axis=1,
                      keepdims=True)
        sel_cols.append(idx)
        val_cols.append(val)
        onehot_sum = onehot_sum + onehot.astype(jnp.float32)
        b = jnp.where(onehot, neg_inf, b)

    sel = jnp.concatenate(sel_cols, axis=1)
    vals = jnp.concatenate(val_cols, axis=1)
    mx = jnp.max(vals, axis=1, keepdims=True)
    e = jnp.exp(vals - mx)
    probs = e / jnp.sum(e, axis=1, keepdims=True)

    sel_ref[...] = sel
    probs_ref[...] = probs

    act = act_ref[...]
    counts_scr[...] += jnp.sum(onehot_sum * act, axis=0, keepdims=True)
    act_scr[...] += jnp.sum(act, axis=(0, 1), keepdims=True)

    @pl.when(i == nsteps - 1)
    def _finish():
        counts = counts_scr[...]
        total = act_scr[...] * jnp.float32(_K)
        freqs = counts / total
        loss_ref[...] = jnp.sum(bias * freqs, axis=1, keepdims=True)
        vio_ref[...] = jnp.float32(L) * jnp.max(freqs - 1.0 / L, axis=1,
                                                keepdims=True)


def kernel(x, active_mask, W_r, expert_bias):
    Bb, Nn, Hh = x.shape
    L = W_r.shape[1]
    BN = Bb * Nn
    T = 1024
    xf = x.reshape(BN, Hh)
    act = active_mask.reshape(BN, 1).astype(jnp.float32)
    bias2 = expert_bias.reshape(1, L)

    out_shape = [
        jax.ShapeDtypeStruct((BN, _K), jnp.int32),
        jax.ShapeDtypeStruct((BN, _K), jnp.float32),
        jax.ShapeDtypeStruct((1, 1), jnp.float32),
        jax.ShapeDtypeStruct((1, 1), jnp.float32),
    ]
    sel, probs, loss, vio = pl.pallas_call(
        _router_kernel,
        grid=(BN // T,),
        in_specs=[
            pl.BlockSpec((T, Hh), lambda i: (i, 0)),
            pl.BlockSpec((Hh, L), lambda i: (0, 0)),
            pl.BlockSpec((1, L), lambda i: (0, 0)),
            pl.BlockSpec((T, 1), lambda i: (i, 0)),
        ],
        out_specs=[
            pl.BlockSpec((T, _K), lambda i: (i, 0)),
            pl.BlockSpec((T, _K), lambda i: (i, 0)),
            pl.BlockSpec((1, 1), lambda i: (0, 0)),
            pl.BlockSpec((1, 1), lambda i: (0, 0)),
        ],
        out_shape=out_shape,
        scratch_shapes=[
            pltpu.VMEM((1, L), jnp.float32),
            pltpu.VMEM((1, 1), jnp.float32),
        ],
    )(xf, W_r, bias2, act)

    return (sel.reshape(Bb, Nn, _K), probs.reshape(Bb, Nn, _K),
            loss[0, 0], vio[0, 0])


# speedup vs baseline: 8.1233x; 1.6711x over previous
import jax
import jax.numpy as jnp
from jax.experimental import pallas as pl
from jax.experimental.pallas import tpu as pltpu

_K = 8


def _router_kernel(x_ref, wt_ref, bias_ref, act_ref,
                   sel_ref, probs_ref, loss_ref, vio_ref,
                   counts_scr, act_scr):
    i = pl.program_id(0)
    nsteps = pl.num_programs(0)

    @pl.when(i == 0)
    def _init():
        counts_scr[...] = jnp.zeros_like(counts_scr)
        act_scr[...] = jnp.zeros_like(act_scr)

    x = x_ref[...]
    wt = wt_ref[...]
    logits = jax.lax.dot_general(wt, x, (((1,), (1,)), ((), ())),
                                 preferred_element_type=jnp.float32)
    bias = bias_ref[...]
    L, T = logits.shape
    iota = jax.lax.broadcasted_iota(jnp.int32, (L, T), 0)
    neg_inf = jnp.float32(-jnp.inf)

    b = logits + bias
    sel_rows = []
    val_rows = []
    onehot_sum = jnp.zeros((L, T), jnp.float32)
    for _ in range(_K):
        m = jnp.max(b, axis=0, keepdims=True)
        idx = jnp.min(jnp.where(b == m, iota, L), axis=0, keepdims=True)
        onehot = iota == idx
        val = jnp.max(jnp.where(onehot, logits, neg_inf), axis=0,
                      keepdims=True)
        sel_rows.append(idx)
        val_rows.append(val)
        onehot_sum = onehot_sum + onehot.astype(jnp.float32)
        b = jnp.where(onehot, neg_inf, b)

    sel = jnp.concatenate(sel_rows, axis=0)
    vals = jnp.concatenate(val_rows, axis=0)
    mx = jnp.max(vals, axis=0, keepdims=True)
    e = jnp.exp(vals - mx)
    probs = e / jnp.sum(e, axis=0, keepdims=True)

    sel_ref[...] = sel
    probs_ref[...] = probs

    act = act_ref[...]
    counts_scr[...] += jnp.sum(onehot_sum * act, axis=1, keepdims=True)
    act_scr[...] += jnp.sum(act, axis=(0, 1), keepdims=True)

    @pl.when(i == nsteps - 1)
    def _finish():
        counts = counts_scr[...]
        total = act_scr[...] * jnp.float32(_K)
        freqs = counts / total
        loss_ref[...] = jnp.sum(bias * freqs, axis=0, keepdims=True)
        vio_ref[...] = jnp.float32(L) * jnp.max(freqs - 1.0 / L, axis=0,
                                                keepdims=True)


def kernel(x, active_mask, W_r, expert_bias):
    Bb, Nn, Hh = x.shape
    L = W_r.shape[1]
    BN = Bb * Nn
    T = 1024
    xf = x.reshape(BN, Hh)
    wt = W_r.T
    act = active_mask.reshape(1, BN).astype(jnp.float32)
    bias2 = expert_bias.reshape(L, 1)

    out_shape = [
        jax.ShapeDtypeStruct((_K, BN), jnp.int32),
        jax.ShapeDtypeStruct((_K, BN), jnp.float32),
        jax.ShapeDtypeStruct((1, 1), jnp.float32),
        jax.ShapeDtypeStruct((1, 1), jnp.float32),
    ]
    sel, probs, loss, vio = pl.pallas_call(
        _router_kernel,
        grid=(BN // T,),
        in_specs=[
            pl.BlockSpec((T, Hh), lambda i: (i, 0)),
            pl.BlockSpec((L, Hh), lambda i: (0, 0)),
            pl.BlockSpec((L, 1), lambda i: (0, 0)),
            pl.BlockSpec((1, T), lambda i: (0, i)),
        ],
        out_specs=[
            pl.BlockSpec((_K, T), lambda i: (0, i)),
            pl.BlockSpec((_K, T), lambda i: (0, i)),
            pl.BlockSpec((1, 1), lambda i: (0, 0)),
            pl.BlockSpec((1, 1), lambda i: (0, 0)),
        ],
        out_shape=out_shape,
        scratch_shapes=[
            pltpu.VMEM((L, 1), jnp.float32),
            pltpu.VMEM((1, 1), jnp.float32),
        ],
    )(xf, wt, bias2, act)

    return (sel.T.reshape(Bb, Nn, _K), probs.T.reshape(Bb, Nn, _K),
            loss[0, 0], vio[0, 0])
